# baseline (speedup 1.0000x reference)
import functools

import jax
import jax.numpy as jnp
from jax import lax
from jax.experimental import pallas as pl
from jax.experimental.pallas import tpu as pltpu
from jax.experimental.pallas import tpu_sc as plsc

_EMBED = 64
_PAIR = 2 * _EMBED
_CHUNK = 128


@functools.lru_cache(maxsize=None)
def _make_gather(n_workers: int, n_chunks: int, vocab_pairs: int):
    b_per_w = n_chunks * _CHUNK
    total = n_workers * b_per_w
    mesh = plsc.VectorSubcoreMesh(core_axis_name="c", subcore_axis_name="s")

    @functools.partial(
        pl.kernel,
        mesh=mesh,
        out_type=jax.ShapeDtypeStruct((total, _PAIR), jnp.float32),
        scratch_types=[
            pltpu.VMEM((n_chunks, _CHUNK), jnp.int32),
            pltpu.VMEM((_CHUNK, _PAIR), jnp.float32),
            pltpu.VMEM((_CHUNK, _PAIR), jnp.float32),
            pltpu.SemaphoreType.DMA,
            pltpu.SemaphoreType.DMA,
        ],
    )
    def gather(idx_hbm, table_hbm, out_hbm, idx_v, rows0, rows1, sem0, sem1):
        n_cores = 2
        wid = lax.axis_index("s") * n_cores + lax.axis_index("c")
        base = wid * b_per_w
        pltpu.sync_copy(idx_hbm.at[wid], idx_v)

        def body(jj, carry):
            j0 = jj * 2
            j1 = j0 + 1
            cp0 = pltpu.async_copy(table_hbm.at[idx_v.at[j0]], rows0, sem0)
            cp1 = pltpu.async_copy(table_hbm.at[idx_v.at[j1]], rows1, sem1)
            cp0.wait()
            pltpu.sync_copy(rows0, out_hbm.at[pl.ds(base + j0 * _CHUNK, _CHUNK)])
            cp1.wait()
            pltpu.sync_copy(rows1, out_hbm.at[pl.ds(base + j1 * _CHUNK, _CHUNK)])
            return carry

        lax.fori_loop(0, n_chunks // 2, body, 0)
        if n_chunks % 2:
            j0 = n_chunks - 1
            pltpu.async_copy(table_hbm.at[idx_v.at[j0]], rows0, sem0).wait()
            pltpu.sync_copy(rows0, out_hbm.at[pl.ds(base + j0 * _CHUNK, _CHUNK)])

    return gather


def kernel(x, table):
    batch, seq = x.shape
    vocab, embed = table.shape
    assert embed == _EMBED and vocab % 2 == 0
    total = batch * seq
    n_workers = 32
    assert total % (n_workers * _CHUNK) == 0
    n_chunks = total // (n_workers * _CHUNK)
    xf = x.reshape(-1).astype(jnp.int32)
    pair_idx = (xf >> 1).reshape(n_workers, n_chunks, _CHUNK)
    table2 = table.reshape(vocab // 2, _PAIR)
    pairs = _make_gather(n_workers, n_chunks, vocab // 2)(pair_idx, table2)
    half = (xf & 1)[:, None] == 1
    out = jnp.where(half, pairs[:, _EMBED:], pairs[:, :_EMBED])
    return out.reshape(batch, seq, embed)


# speedup vs baseline: 1.1504x; 1.1504x over previous
import functools

import jax
import jax.numpy as jnp
from jax import lax
from jax.experimental import pallas as pl
from jax.experimental.pallas import tpu as pltpu
from jax.experimental.pallas import tpu_sc as plsc

_EMBED = 64
_K = 8


@functools.lru_cache(maxsize=None)
def _make_gather(n_workers: int, b_per_w: int):
    total = n_workers * b_per_w
    mesh = plsc.VectorSubcoreMesh(core_axis_name="c", subcore_axis_name="s")

    @functools.partial(
        pl.kernel,
        mesh=mesh,
        out_type=jax.ShapeDtypeStruct((total, _EMBED), jnp.float32),
        scratch_types=[
            pltpu.VMEM((b_per_w,), jnp.int32),
            pltpu.VMEM((16, _EMBED), jnp.float32),
            pltpu.SemaphoreType.DMA,
        ],
    )
    def gather(idx_hbm, table_hbm, out_hbm, idx_v, rows_v, sem):
        n_cores = 2
        wid = lax.axis_index("s") * n_cores + lax.axis_index("c")
        base = wid * b_per_w
        pltpu.sync_copy(idx_hbm.at[wid], idx_v)

        def body(g, carry):
            vec = idx_v[pl.ds(g * 16, 16)]
            cps = [
                pltpu.async_copy(table_hbm.at[vec[l]], rows_v.at[l], sem)
                for l in range(16)
            ]
            for cp in cps:
                cp.wait()
            pltpu.sync_copy(rows_v, out_hbm.at[pl.ds(base + g * 16, 16)])
            return carry

        lax.fori_loop(0, b_per_w // 16, body, 0)

    return gather


def kernel(x, table):
    batch, seq = x.shape
    vocab, embed = table.shape
    total = batch * seq
    n_workers = 32
    b_per_w = total // n_workers
    xf = x.reshape(n_workers, b_per_w).astype(jnp.int32)
    out = _make_gather(n_workers, b_per_w)(xf, table)
    return out.reshape(batch, seq, embed)


# speedup vs baseline: 1.6843x; 1.4640x over previous
import functools

import jax
import jax.numpy as jnp
from jax import lax
from jax.experimental import pallas as pl
from jax.experimental.pallas import tpu as pltpu
from jax.experimental.pallas import tpu_sc as plsc

_EMBED = 64
_G = 128


@functools.lru_cache(maxsize=None)
def _make_gather(n_workers: int, b_per_w: int):
    total = n_workers * b_per_w
    n_groups = b_per_w // _G
    mesh = plsc.VectorSubcoreMesh(core_axis_name="c", subcore_axis_name="s")

    @functools.partial(
        pl.kernel,
        mesh=mesh,
        out_type=jax.ShapeDtypeStruct((total, _EMBED), jnp.float32),
        scratch_types=[
            pltpu.VMEM((b_per_w,), jnp.int32),
            pltpu.VMEM((_G, _EMBED), jnp.float32),
            pltpu.VMEM((_G, _EMBED), jnp.float32),
            pltpu.SemaphoreType.DMA,
            pltpu.SemaphoreType.DMA,
            pltpu.SemaphoreType.DMA,
            pltpu.SemaphoreType.DMA,
        ],
    )
    def gather(idx_hbm, table_hbm, out_hbm, idx_v, rb0, rb1,
               rsem0, rsem1, wsem0, wsem1):
        n_cores = 2
        wid = lax.axis_index("s") * n_cores + lax.axis_index("c")
        base = wid * b_per_w
        pltpu.sync_copy(idx_hbm.at[wid], idx_v)

        def fire(g, rb, rsem):
            def sub(k, c):
                vec = idx_v[pl.ds(g * _G + k * 16, 16)]
                for l in range(16):
                    pltpu.async_copy(table_hbm.at[vec[l]], rb.at[k * 16 + l], rsem)
                return c
            lax.fori_loop(0, _G // 16, sub, 0)

        def drain_reads(rb, rsem):
            pltpu.make_async_copy(table_hbm.at[pl.ds(0, _G)], rb, rsem).wait()

        def write(g, rb, wsem):
            pltpu.async_copy(rb, out_hbm.at[pl.ds(base + g * _G, _G)], wsem)

        def wait_write(rb, wsem):
            pltpu.make_async_copy(table_hbm.at[pl.ds(0, _G)], rb, wsem).wait()

        fire(0, rb0, rsem0)
        fire(1, rb1, rsem1)

        def body(gg, carry):
            a = gg * 2
            drain_reads(rb0, rsem0)
            write(a, rb0, wsem0)
            drain_reads(rb1, rsem1)
            write(a + 1, rb1, wsem1)

            @pl.when(gg < n_groups // 2 - 1)
            def _():
                wait_write(rb0, wsem0)
                fire(a + 2, rb0, rsem0)
                wait_write(rb1, wsem1)
                fire(a + 3, rb1, rsem1)
            return carry

        lax.fori_loop(0, n_groups // 2, body, 0)
        wait_write(rb0, wsem0)
        wait_write(rb1, wsem1)

    return gather


def kernel(x, table):
    batch, seq = x.shape
    vocab, embed = table.shape
    assert embed == _EMBED
    total = batch * seq
    n_workers = 32
    b_per_w = total // n_workers
    assert b_per_w % (2 * _G) == 0
    xf = x.reshape(n_workers, b_per_w).astype(jnp.int32)
    out = _make_gather(n_workers, b_per_w)(xf, table)
    return out.reshape(batch, seq, embed)


# speedup vs baseline: 1.9338x; 1.1481x over previous
import functools

import jax
import jax.numpy as jnp
from jax import lax
from jax.experimental import pallas as pl
from jax.experimental.pallas import tpu as pltpu
from jax.experimental.pallas import tpu_sc as plsc

_EMBED = 64
_GB = 8


@functools.lru_cache(maxsize=None)
def _make_gather(batch: int, seq: int):
    n_workers = 32
    bat_per_w = batch // n_workers
    b_per_w = bat_per_w * seq
    g_rows = _GB * seq
    n_groups = bat_per_w // _GB
    mesh = plsc.VectorSubcoreMesh(core_axis_name="c", subcore_axis_name="s")

    @functools.partial(
        pl.kernel,
        mesh=mesh,
        out_type=jax.ShapeDtypeStruct((batch, seq, _EMBED), jnp.float32),
        scratch_types=[
            pltpu.VMEM((b_per_w,), jnp.int32),
            pltpu.VMEM((_GB, seq, _EMBED), jnp.float32),
            pltpu.VMEM((_GB, seq, _EMBED), jnp.float32),
            pltpu.SemaphoreType.DMA,
            pltpu.SemaphoreType.DMA,
            pltpu.SemaphoreType.DMA,
            pltpu.SemaphoreType.DMA,
        ],
        compiler_params=pltpu.CompilerParams(use_tc_tiling_on_sc=True),
    )
    def gather(idx_hbm, table_hbm, out_hbm, idx_v, rb0, rb1,
               rsem0, rsem1, wsem0, wsem1):
        n_cores = 2
        wid = lax.axis_index("s") * n_cores + lax.axis_index("c")
        bat_base = wid * bat_per_w
        pltpu.sync_copy(idx_hbm.at[wid], idx_v)

        def fire(g, rb, rsem):
            def sub(k, c):
                j0 = k * 16
                vec = idx_v[pl.ds(g * g_rows + j0, 16)]
                for l in range(16):
                    j = j0 + l
                    q = j // seq
                    t = j % seq
                    pltpu.async_copy(table_hbm.at[vec[l]], rb.at[q, t], rsem)
                return c
            lax.fori_loop(0, g_rows // 16, sub, 0)

        def drain_reads(rb, rsem):
            pltpu.make_async_copy(out_hbm.at[pl.ds(0, _GB)], rb, rsem).wait()

        def write(g, rb, wsem):
            pltpu.async_copy(rb, out_hbm.at[pl.ds(bat_base + g * _GB, _GB)], wsem)

        def wait_write(rb, wsem):
            pltpu.make_async_copy(out_hbm.at[pl.ds(0, _GB)], rb, wsem).wait()

        fire(0, rb0, rsem0)
        fire(1, rb1, rsem1)

        def body(gg, carry):
            a = gg * 2
            drain_reads(rb0, rsem0)
            write(a, rb0, wsem0)
            drain_reads(rb1, rsem1)
            write(a + 1, rb1, wsem1)

            @pl.when(gg < n_groups // 2 - 1)
            def _():
                wait_write(rb0, wsem0)
                fire(a + 2, rb0, rsem0)
                wait_write(rb1, wsem1)
                fire(a + 3, rb1, rsem1)
            return carry

        lax.fori_loop(0, n_groups // 2, body, 0)
        wait_write(rb0, wsem0)
        wait_write(rb1, wsem1)

    return gather


def kernel(x, table):
    batch, seq = x.shape
    vocab, embed = table.shape
    assert embed == _EMBED
    n_workers = 32
    assert batch % n_workers == 0 and (batch // n_workers) % _GB == 0
    xf = x.reshape(n_workers, (batch // n_workers) * seq).astype(jnp.int32)
    return _make_gather(batch, seq)(xf, table)


# speedup vs baseline: 2.1755x; 1.1250x over previous
import functools

import jax
import jax.numpy as jnp
from jax import lax
from jax.experimental import pallas as pl
from jax.experimental.pallas import tpu as pltpu
from jax.experimental.pallas import tpu_sc as plsc

_EMBED = 64
_GB = 8


@functools.lru_cache(maxsize=None)
def _make_gather(batch: int, seq: int):
    n_workers = 32
    bat_per_w = batch // n_workers
    b_per_w = bat_per_w * seq
    g_rows = _GB * seq
    n_groups = bat_per_w // _GB
    seq2 = seq // 2
    mesh = plsc.VectorSubcoreMesh(core_axis_name="c", subcore_axis_name="s")

    @functools.partial(
        pl.kernel,
        mesh=mesh,
        out_type=jax.ShapeDtypeStruct((batch, seq2, 2 * _EMBED), jnp.float32),
        scratch_types=[
            pltpu.VMEM((b_per_w,), jnp.int32),
            pltpu.VMEM((_GB, seq2, 2 * _EMBED), jnp.float32),
            pltpu.VMEM((_GB, seq2, 2 * _EMBED), jnp.float32),
            pltpu.SemaphoreType.DMA,
            pltpu.SemaphoreType.DMA,
            pltpu.SemaphoreType.DMA,
            pltpu.SemaphoreType.DMA,
        ],
        compiler_params=pltpu.CompilerParams(use_tc_tiling_on_sc=True),
    )
    def gather(idx_hbm, table_hbm, out_hbm, idx_v, rb0, rb1,
               rsem0, rsem1, wsem0, wsem1):
        n_cores = 2
        wid = lax.axis_index("s") * n_cores + lax.axis_index("c")
        bat_base = wid * bat_per_w
        pltpu.sync_copy(idx_hbm.at[wid], idx_v)

        starts_lanes = [(0, range(16)), (16, range(16)), (32, range(16)),
                        (seq - 16, range(14, 16))]

        def fire(g, rb, rsem):
            def one_batch(q, c):
                qbase = g * g_rows + q * seq
                for j0, lanes in starts_lanes:
                    vec = idx_v[pl.ds(qbase + j0, 16)]
                    for l in lanes:
                        t = j0 + l
                        src = table_hbm.at[vec[l]]
                        dst = rb.at[q, t // 2, pl.ds((t % 2) * _EMBED, _EMBED)]
                        pltpu.async_copy(src, dst, rsem)
                return c
            lax.fori_loop(0, _GB, one_batch, 0)

        def drain_reads(rb, rsem):
            pltpu.make_async_copy(out_hbm.at[pl.ds(0, _GB)], rb, rsem).wait()

        def write(g, rb, wsem):
            pltpu.async_copy(rb, out_hbm.at[pl.ds(bat_base + g * _GB, _GB)], wsem)

        def wait_write(rb, wsem):
            pltpu.make_async_copy(out_hbm.at[pl.ds(0, _GB)], rb, wsem).wait()

        fire(0, rb0, rsem0)
        fire(1, rb1, rsem1)

        def body(gg, carry):
            a = gg * 2
            drain_reads(rb0, rsem0)
            write(a, rb0, wsem0)
            drain_reads(rb1, rsem1)
            write(a + 1, rb1, wsem1)

            @pl.when(gg < n_groups // 2 - 1)
            def _():
                wait_write(rb0, wsem0)
                fire(a + 2, rb0, rsem0)
                wait_write(rb1, wsem1)
                fire(a + 3, rb1, rsem1)
            return carry

        lax.fori_loop(0, n_groups // 2, body, 0)
        wait_write(rb0, wsem0)
        wait_write(rb1, wsem1)

    return gather


def kernel(x, table):
    batch, seq = x.shape
    vocab, embed = table.shape
    assert embed == _EMBED and seq % 2 == 0 and vocab % 2 == 0
    n_workers = 32
    assert batch % n_workers == 0 and (batch // n_workers) % _GB == 0
    xf = x.reshape(n_workers, (batch // n_workers) * seq).astype(jnp.int32)
    out = _make_gather(batch, seq)(xf, table)
    return out.reshape(batch, seq, embed)
